# baseline (speedup 1.0000x reference)
import jax
import jax.numpy as jnp
from jax import lax
from jax.experimental import pallas as pl
from jax.experimental.pallas import tpu as pltpu
from jax.experimental.pallas import tpu_sc as plsc

VOCAB = 1_000_000
DIM = 64
BATCH = 16384
NUM_CORES = 2
NUM_SUBCORES = 16
NUM_WORKERS = NUM_CORES * NUM_SUBCORES
BPW = BATCH // NUM_WORKERS
CHUNK = 128
NCHUNKS = BPW // CHUNK
GROUPS_PER_CHUNK = CHUNK // 16


def _emb_body(table_hbm, qidx_hbm, coloff_hbm, out_hbm,
              qidx_v, coloff_v, rows_v, outbuf, g0, g1, g2, g3, ssem):
    gsems = (g0, g1, g2, g3)
    wid = lax.axis_index("s") * NUM_CORES + lax.axis_index("c")
    pltpu.sync_copy(qidx_hbm.at[pl.ds(wid, 1)], qidx_v)
    pltpu.sync_copy(coloff_hbm.at[pl.ds(wid * BPW, BPW)], coloff_v)
    gathers = [
        pltpu.async_copy(
            table_hbm.at[qidx_v.at[0, j]],
            rows_v.at[pl.ds(j * CHUNK, CHUNK)],
            gsems[j],
        )
        for j in range(NCHUNKS)
    ]
    iota = lax.iota(jnp.int32, 16)

    def select_group(i, carry):
        colb = coloff_v[pl.ds(i * 16, 16)]
        rowv = i * 16 + iota
        dstb = rowv * DIM
        for w in range(DIM):
            x = plsc.load_gather(rows_v, [rowv, colb + w])
            plsc.store_scatter(outbuf, [dstb + w], x)
        return carry

    for j in range(NCHUNKS):
        gathers[j].wait()
        lax.fori_loop(j * GROUPS_PER_CHUNK, (j + 1) * GROUPS_PER_CHUNK,
                      select_group, 0)
    pltpu.sync_copy(outbuf, out_hbm.at[pl.ds(wid * BPW * DIM, BPW * DIM)])


def kernel(user_id, table):
    idx = user_id.astype(jnp.int32)
    table_pairs = table.reshape(VOCAB // 2, 2 * DIM)
    qidx = (idx >> 1).reshape(NUM_WORKERS, NCHUNKS, CHUNK)
    coloff = (idx & 1) << 6
    mesh = plsc.VectorSubcoreMesh(core_axis_name="c", subcore_axis_name="s")
    run = pl.kernel(
        _emb_body,
        mesh=mesh,
        out_type=jax.ShapeDtypeStruct((BATCH * DIM,), jnp.float32),
        scratch_types=[
            pltpu.VMEM((1, NCHUNKS, CHUNK), jnp.int32),
            pltpu.VMEM((BPW,), jnp.int32),
            pltpu.VMEM((BPW, 2 * DIM), jnp.float32),
            pltpu.VMEM((BPW * DIM,), jnp.float32),
            pltpu.SemaphoreType.DMA,
            pltpu.SemaphoreType.DMA,
            pltpu.SemaphoreType.DMA,
            pltpu.SemaphoreType.DMA,
            pltpu.SemaphoreType.DMA,
        ],
        compiler_params=pltpu.CompilerParams(
            use_tc_tiling_on_sc=False, needs_layout_passes=False),
    )
    return run(table_pairs, qidx, coloff).reshape(BATCH, DIM)
